# baseline (speedup 1.0000x reference)
import functools

import jax
import jax.numpy as jnp
from jax import lax
from jax.experimental import pallas as pl
from jax.experimental.pallas import tpu as pltpu
from jax.experimental.pallas import tpu_sc as plsc

N_NODES = 10000
N_EDGES = 320000
D = 128

NC, NS = 2, 16
NW = NC * NS
CHUNK = 128

GF, GS = 160, 0
FAST_CID = 0
N_CHUNKS = NS * (GF + GS)
CHUNKS_PAD = N_CHUNKS + 64
E_PAD = CHUNKS_PAD * CHUNK
NPH = 4
GBUF = GF // NPH

GD = N_CHUNKS // NW

ACC_ROWS = 10112
ZBLK = ACC_ROWS // NS
DEG_W = 16

_mesh = plsc.VectorSubcoreMesh(core_axis_name="c", subcore_axis_name="s")


@functools.partial(
    pl.kernel,
    out_type=jax.ShapeDtypeStruct((NC, ACC_ROWS, DEG_W), jnp.float32),
    mesh=_mesh,
    scratch_types=[
        pltpu.VMEM((GD, CHUNK), jnp.int32),
        pltpu.VMEM((CHUNK, DEG_W), jnp.float32),
        pltpu.VMEM_SHARED((ACC_ROWS, DEG_W), jnp.float32),
    ],
)
def _deg_kernel(dst_hbm, zeros_hbm, out_hbm, idx_v, ones_v, acc):
    cid = lax.axis_index("c")
    sid = lax.axis_index("s")
    wid = cid * NS + sid
    pltpu.sync_copy(zeros_hbm.at[pl.ds(sid * ZBLK, ZBLK)],
                    acc.at[pl.ds(sid * ZBLK, ZBLK)])
    pltpu.sync_copy(dst_hbm.at[pl.ds(wid * GD, GD)], idx_v)

    def fill(i, carry):
        ones_v[i, :] = jnp.full((DEG_W,), 1.0, jnp.float32)
        return carry

    lax.fori_loop(0, CHUNK, fill, 0)
    plsc.subcore_barrier()

    def body(g, carry):
        pltpu.sync_copy(ones_v, acc.at[idx_v.at[g]], add=True)
        return carry

    lax.fori_loop(0, GD, body, 0)
    plsc.subcore_barrier()
    pltpu.sync_copy(acc.at[pl.ds(sid * ZBLK, ZBLK)],
                    out_hbm.at[cid, pl.ds(sid * ZBLK, ZBLK)])


@functools.partial(
    pl.kernel,
    out_type=jax.ShapeDtypeStruct((NC, ACC_ROWS, D), jnp.float32),
    mesh=_mesh,
    scratch_types=[
        pltpu.VMEM((GBUF, CHUNK), jnp.int32),
        pltpu.VMEM((GBUF, CHUNK), jnp.int32),
        pltpu.VMEM((2, CHUNK, D), jnp.float32),
        pltpu.VMEM_SHARED((ACC_ROWS, D), jnp.float32),
        pltpu.SemaphoreType.DMA,
        pltpu.SemaphoreType.DMA,
    ],
)
def _msg_kernel(src_hbm, dst_hbm, z_hbm, zeros_hbm, out_hbm,
                isrc, idst, rows, acc, sem_g, sem_s):
    cid = lax.axis_index("c")
    sid = lax.axis_index("s")
    fast = cid == FAST_CID
    half = jnp.where(fast, GF // NPH, GS // NPH)
    chunk0 = jnp.where(fast, sid * GF, NS * GF + sid * GS)
    pltpu.sync_copy(zeros_hbm.at[pl.ds(sid * ZBLK, ZBLK)],
                    acc.at[pl.ds(sid * ZBLK, ZBLK)])
    plsc.subcore_barrier()

    @pl.when(half > 0)
    def _work():
        for p in range(NPH):
            start = chunk0 + p * half
            pltpu.sync_copy(src_hbm.at[pl.ds(start, GBUF)], isrc)
            pltpu.sync_copy(dst_hbm.at[pl.ds(start, GBUF)], idst)

            def body(i, carry):
                g0 = 2 * i
                ca = pltpu.async_copy(z_hbm.at[isrc.at[g0]],
                                      rows.at[0], sem_g)
                cb = pltpu.async_copy(z_hbm.at[isrc.at[g0 + 1]],
                                      rows.at[1], sem_g)
                ca.wait()
                cb.wait()
                sa = pltpu.async_copy(rows.at[0], acc.at[idst.at[g0]],
                                      sem_s, add=True)
                sb = pltpu.async_copy(rows.at[1], acc.at[idst.at[g0 + 1]],
                                      sem_s, add=True)
                sa.wait()
                sb.wait()
                return carry

            lax.fori_loop(0, half // 2, body, 0)

    plsc.subcore_barrier()
    pltpu.sync_copy(acc.at[pl.ds(sid * ZBLK, ZBLK)],
                    out_hbm.at[cid, pl.ds(sid * ZBLK, ZBLK)])


def _z_body(x_ref, w_ref, d0_ref, d1_ref, z_ref, dinv_ref):
    deg = d0_ref[...] + d1_ref[...] + 1.0
    dinv = lax.rsqrt(deg)
    xw = jnp.dot(x_ref[...], w_ref[...], preferred_element_type=jnp.float32)
    z_ref[...] = xw * dinv
    dinv_ref[...] = dinv


def _final_body(a0_ref, a1_ref, z_ref, dinv_ref, b_ref, out_ref):
    s = a0_ref[...] + a1_ref[...] + z_ref[...]
    out_ref[...] = s * dinv_ref[...] + b_ref[...]


def _emlp_body(e_ref, w_ref, b_ref, out_ref):
    out_ref[...] = (
        jnp.dot(e_ref[...], w_ref[...], preferred_element_type=jnp.float32)
        + b_ref[...]
    )


def kernel(X, E, emb_nodes, emb_edges, edge_index, W_conv, b_conv, W_e, b_e):
    src = edge_index[0]
    dst = edge_index[1]
    pad = E_PAD - N_EDGES
    src_p = jnp.concatenate(
        [src, jnp.zeros((pad,), jnp.int32)]).reshape(CHUNKS_PAD, CHUNK)
    dst_p = jnp.concatenate(
        [dst, jnp.full((pad,), N_NODES, jnp.int32)]).reshape(CHUNKS_PAD, CHUNK)
    zdeg = jnp.zeros((ACC_ROWS, DEG_W), jnp.float32)
    znd = jnp.zeros((ACC_ROWS, D), jnp.float32)

    degp = _deg_kernel(dst_p, zdeg)
    d0 = degp[0, :N_NODES, 0:1]
    d1 = degp[1, :N_NODES, 0:1]

    BR = 2000
    Z, dinv = pl.pallas_call(
        _z_body,
        grid=(N_NODES // BR,),
        in_specs=[
            pl.BlockSpec((BR, D), lambda i: (i, 0)),
            pl.BlockSpec((D, D), lambda i: (0, 0)),
            pl.BlockSpec((BR, 1), lambda i: (i, 0)),
            pl.BlockSpec((BR, 1), lambda i: (i, 0)),
        ],
        out_specs=[
            pl.BlockSpec((BR, D), lambda i: (i, 0)),
            pl.BlockSpec((BR, 1), lambda i: (i, 0)),
        ],
        out_shape=[
            jax.ShapeDtypeStruct((N_NODES, D), jnp.float32),
            jax.ShapeDtypeStruct((N_NODES, 1), jnp.float32),
        ],
    )(X, W_conv, d0, d1)

    accp = _msg_kernel(src_p, dst_p, Z, znd)
    a0 = accp[0, :N_NODES]
    a1 = accp[1, :N_NODES]

    X_new = pl.pallas_call(
        _final_body,
        grid=(N_NODES // BR,),
        in_specs=[
            pl.BlockSpec((BR, D), lambda i: (i, 0)),
            pl.BlockSpec((BR, D), lambda i: (i, 0)),
            pl.BlockSpec((BR, D), lambda i: (i, 0)),
            pl.BlockSpec((BR, 1), lambda i: (i, 0)),
            pl.BlockSpec((1, D), lambda i: (0, 0)),
        ],
        out_specs=pl.BlockSpec((BR, D), lambda i: (i, 0)),
        out_shape=jax.ShapeDtypeStruct((N_NODES, D), jnp.float32),
    )(a0, a1, Z, dinv, b_conv.reshape(1, D))

    BE = 2000
    E_new = pl.pallas_call(
        _emlp_body,
        grid=(N_EDGES // BE,),
        in_specs=[
            pl.BlockSpec((BE, D), lambda i: (i, 0)),
            pl.BlockSpec((D, D), lambda i: (0, 0)),
            pl.BlockSpec((1, D), lambda i: (0, 0)),
        ],
        out_specs=pl.BlockSpec((BE, D), lambda i: (i, 0)),
        out_shape=jax.ShapeDtypeStruct((N_EDGES, D), jnp.float32),
    )(E, W_e, b_e.reshape(1, D))

    return (X_new, E_new, X)


# speedup vs baseline: 1.2951x; 1.2951x over previous
import functools

import jax
import jax.numpy as jnp
from jax import lax
from jax.experimental import pallas as pl
from jax.experimental.pallas import tpu as pltpu
from jax.experimental.pallas import tpu_sc as plsc

N_NODES = 10000
N_EDGES = 320000
D = 128

NC, NS = 2, 16
NW = NC * NS
CHUNK = 128

GF, GS = 144, 16
FAST_CID = 0
N_CHUNKS = NS * (GF + GS)
CHUNKS_PAD = N_CHUNKS + 64
E_PAD = CHUNKS_PAD * CHUNK
NPH = 3
GBUF = GF // NPH

GD = N_CHUNKS // NW

ACC_ROWS = 10112
ZBLK = ACC_ROWS // NS
DEG_W = 16

_mesh = plsc.VectorSubcoreMesh(core_axis_name="c", subcore_axis_name="s")


@functools.partial(
    pl.kernel,
    out_type=jax.ShapeDtypeStruct((NC, ACC_ROWS, DEG_W), jnp.float32),
    mesh=_mesh,
    scratch_types=[
        pltpu.VMEM((GD, CHUNK), jnp.int32),
        pltpu.VMEM((CHUNK, DEG_W), jnp.float32),
        pltpu.VMEM_SHARED((ACC_ROWS, DEG_W), jnp.float32),
    ],
)
def _deg_kernel(dst_hbm, zeros_hbm, out_hbm, idx_v, ones_v, acc):
    cid = lax.axis_index("c")
    sid = lax.axis_index("s")
    wid = cid * NS + sid
    pltpu.sync_copy(zeros_hbm.at[pl.ds(sid * ZBLK, ZBLK)],
                    acc.at[pl.ds(sid * ZBLK, ZBLK)])
    pltpu.sync_copy(dst_hbm.at[pl.ds(wid * GD, GD)], idx_v)

    def fill(i, carry):
        ones_v[i, :] = jnp.full((DEG_W,), 1.0, jnp.float32)
        return carry

    lax.fori_loop(0, CHUNK, fill, 0)
    plsc.subcore_barrier()

    def body(g, carry):
        pltpu.sync_copy(ones_v, acc.at[idx_v.at[g]], add=True)
        return carry

    lax.fori_loop(0, GD, body, 0)
    plsc.subcore_barrier()
    pltpu.sync_copy(acc.at[pl.ds(sid * ZBLK, ZBLK)],
                    out_hbm.at[cid, pl.ds(sid * ZBLK, ZBLK)])


@functools.partial(
    pl.kernel,
    out_type=jax.ShapeDtypeStruct((NC, ACC_ROWS, D), jnp.float32),
    mesh=_mesh,
    scratch_types=[
        pltpu.VMEM((GBUF, CHUNK), jnp.int32),
        pltpu.VMEM((GBUF, CHUNK), jnp.int32),
        pltpu.VMEM((2, CHUNK, D), jnp.float32),
        pltpu.VMEM_SHARED((ACC_ROWS, D), jnp.float32),
        pltpu.SemaphoreType.DMA,
        pltpu.SemaphoreType.DMA,
    ],
)
def _msg_kernel(src_hbm, dst_hbm, z_hbm, zeros_hbm, out_hbm,
                isrc, idst, rows, acc, sem_g, sem_s):
    cid = lax.axis_index("c")
    sid = lax.axis_index("s")
    fast = cid == FAST_CID
    half = jnp.where(fast, GF // NPH, GS)
    chunk0 = jnp.where(fast, sid * GF, NS * GF + sid * GS)
    pltpu.sync_copy(zeros_hbm.at[pl.ds(sid * ZBLK, ZBLK)],
                    acc.at[pl.ds(sid * ZBLK, ZBLK)])
    plsc.subcore_barrier()

    for p in range(NPH):
        @pl.when((half > 0) if p == 0 else fast)
        def _work():
            start = chunk0 + p * (GF // NPH)
            pltpu.sync_copy(src_hbm.at[pl.ds(start, GBUF)], isrc)
            pltpu.sync_copy(dst_hbm.at[pl.ds(start, GBUF)], idst)

            def body(i, carry):
                g0 = 2 * i
                ca = pltpu.async_copy(z_hbm.at[isrc.at[g0]],
                                      rows.at[0], sem_g)
                cb = pltpu.async_copy(z_hbm.at[isrc.at[g0 + 1]],
                                      rows.at[1], sem_g)
                ca.wait()
                cb.wait()
                sa = pltpu.async_copy(rows.at[0], acc.at[idst.at[g0]],
                                      sem_s, add=True)
                sb = pltpu.async_copy(rows.at[1], acc.at[idst.at[g0 + 1]],
                                      sem_s, add=True)
                sa.wait()
                sb.wait()
                return carry

            lax.fori_loop(0, half // 2, body, 0)

    plsc.subcore_barrier()
    pltpu.sync_copy(acc.at[pl.ds(sid * ZBLK, ZBLK)],
                    out_hbm.at[cid, pl.ds(sid * ZBLK, ZBLK)])


def _z_body(x_ref, w_ref, d0_ref, d1_ref, z_ref, dinv_ref):
    deg = d0_ref[...] + d1_ref[...] + 1.0
    dinv = lax.rsqrt(deg)
    xw = jnp.dot(x_ref[...], w_ref[...], preferred_element_type=jnp.float32)
    z_ref[...] = xw * dinv
    dinv_ref[...] = dinv


def _final_body(a0_ref, a1_ref, z_ref, dinv_ref, b_ref, out_ref):
    s = a0_ref[...] + a1_ref[...] + z_ref[...]
    out_ref[...] = s * dinv_ref[...] + b_ref[...]


def _emlp_body(e_ref, w_ref, b_ref, out_ref):
    out_ref[...] = (
        jnp.dot(e_ref[...], w_ref[...], preferred_element_type=jnp.float32)
        + b_ref[...]
    )


def kernel(X, E, emb_nodes, emb_edges, edge_index, W_conv, b_conv, W_e, b_e):
    src = edge_index[0]
    dst = edge_index[1]
    pad = E_PAD - N_EDGES
    src_p = jnp.concatenate(
        [src, jnp.zeros((pad,), jnp.int32)]).reshape(CHUNKS_PAD, CHUNK)
    dst_p = jnp.concatenate(
        [dst, jnp.full((pad,), N_NODES, jnp.int32)]).reshape(CHUNKS_PAD, CHUNK)
    zdeg = jnp.zeros((ACC_ROWS, DEG_W), jnp.float32)
    znd = jnp.zeros((ACC_ROWS, D), jnp.float32)

    degp = _deg_kernel(dst_p, zdeg)
    d0 = degp[0, :N_NODES, 0:1]
    d1 = degp[1, :N_NODES, 0:1]

    BR = 2000
    Z, dinv = pl.pallas_call(
        _z_body,
        grid=(N_NODES // BR,),
        in_specs=[
            pl.BlockSpec((BR, D), lambda i: (i, 0)),
            pl.BlockSpec((D, D), lambda i: (0, 0)),
            pl.BlockSpec((BR, 1), lambda i: (i, 0)),
            pl.BlockSpec((BR, 1), lambda i: (i, 0)),
        ],
        out_specs=[
            pl.BlockSpec((BR, D), lambda i: (i, 0)),
            pl.BlockSpec((BR, 1), lambda i: (i, 0)),
        ],
        out_shape=[
            jax.ShapeDtypeStruct((N_NODES, D), jnp.float32),
            jax.ShapeDtypeStruct((N_NODES, 1), jnp.float32),
        ],
    )(X, W_conv, d0, d1)

    accp = _msg_kernel(src_p, dst_p, Z, znd)
    a0 = accp[0, :N_NODES]
    a1 = accp[1, :N_NODES]

    X_new = pl.pallas_call(
        _final_body,
        grid=(N_NODES // BR,),
        in_specs=[
            pl.BlockSpec((BR, D), lambda i: (i, 0)),
            pl.BlockSpec((BR, D), lambda i: (i, 0)),
            pl.BlockSpec((BR, D), lambda i: (i, 0)),
            pl.BlockSpec((BR, 1), lambda i: (i, 0)),
            pl.BlockSpec((1, D), lambda i: (0, 0)),
        ],
        out_specs=pl.BlockSpec((BR, D), lambda i: (i, 0)),
        out_shape=jax.ShapeDtypeStruct((N_NODES, D), jnp.float32),
    )(a0, a1, Z, dinv, b_conv.reshape(1, D))

    BE = 2000
    E_new = pl.pallas_call(
        _emlp_body,
        grid=(N_EDGES // BE,),
        in_specs=[
            pl.BlockSpec((BE, D), lambda i: (i, 0)),
            pl.BlockSpec((D, D), lambda i: (0, 0)),
            pl.BlockSpec((1, D), lambda i: (0, 0)),
        ],
        out_specs=pl.BlockSpec((BE, D), lambda i: (i, 0)),
        out_shape=jax.ShapeDtypeStruct((N_EDGES, D), jnp.float32),
    )(E, W_e, b_e.reshape(1, D))

    return (X_new, E_new, X)
